# baseline (speedup 1.0000x reference)
import jax
import jax.numpy as jnp
from jax import lax
from jax.experimental import pallas as pl
from jax.experimental.pallas import tpu as pltpu
from jax.experimental.pallas import tpu_sc as plsc

B = 4096
L = 200
DIM = 64

NC = 2
NS = 16
NW = NC * NS
TOTAL = B * L
PER_W = TOTAL // NW
CHUNK = 128
NCHUNK = PER_W // CHUNK
NBUF = 4


def _body(idx_hbm, table_hbm, out_hbm, idx_v, rows_v, gsem, osem):
    cid = lax.axis_index("c")
    sid = lax.axis_index("s")
    wid = sid * NC + cid
    base_w = wid * PER_W

    pltpu.sync_copy(idx_hbm.at[wid], idx_v)

    @pl.loop(0, NCHUNK, step=NBUF)
    def _group(c0):
        for b in range(NBUF):
            pltpu.async_copy(table_hbm.at[idx_v.at[c0 + b]], rows_v.at[b], gsem)
        for b in range(NBUF):
            pltpu.make_async_copy(
                table_hbm.at[idx_v.at[c0 + b]], rows_v.at[b], gsem
            ).wait()
            pltpu.async_copy(
                rows_v.at[b],
                out_hbm.at[pl.ds(base_w + (c0 + b) * CHUNK, CHUNK)],
                osem,
            )
        for b in range(NBUF):
            pltpu.make_async_copy(
                rows_v.at[b],
                out_hbm.at[pl.ds(base_w + (c0 + b) * CHUNK, CHUNK)],
                osem,
            ).wait()


_mesh = plsc.VectorSubcoreMesh(core_axis_name="c", subcore_axis_name="s")

_gather_call = pl.kernel(
    _body,
    out_type=jax.ShapeDtypeStruct((TOTAL, DIM), jnp.float32),
    mesh=_mesh,
    scratch_types=[
        pltpu.VMEM((NCHUNK, CHUNK), jnp.int32),
        pltpu.VMEM((NBUF, CHUNK, DIM), jnp.float32),
        pltpu.SemaphoreType.DMA,
        pltpu.SemaphoreType.DMA,
    ],
    compiler_params=pltpu.CompilerParams(use_tc_tiling_on_sc=False),
)


@jax.jit
def _run(idx3, weight):
    return _gather_call(idx3, weight)


def kernel(gene_indices, weight):
    idx3 = jnp.asarray(gene_indices, jnp.int32).reshape(NW, NCHUNK, CHUNK)
    out = _run(idx3, weight)
    return out.reshape(B, L, DIM)


# speedup vs baseline: 1.0013x; 1.0013x over previous
import jax
import jax.numpy as jnp
from jax import lax
from jax.experimental import pallas as pl
from jax.experimental.pallas import tpu as pltpu
from jax.experimental.pallas import tpu_sc as plsc

B = 4096
L = 200
DIM = 64

NC = 2
NS = 16
NW = NC * NS
TOTAL = B * L
PER_W = TOTAL // NW
CHUNK = 512
NCHUNK = PER_W // CHUNK
NBUF = 2


def _body(idx_hbm, table_hbm, out_hbm, idx_v, rows_v, gsem, osem):
    cid = lax.axis_index("c")
    sid = lax.axis_index("s")
    wid = sid * NC + cid
    base_w = wid * PER_W

    pltpu.sync_copy(idx_hbm.at[wid], idx_v)

    @pl.loop(0, NCHUNK, step=NBUF)
    def _group(c0):
        for b in range(NBUF):
            pltpu.async_copy(table_hbm.at[idx_v.at[c0 + b]], rows_v.at[b], gsem)
        for b in range(NBUF):
            pltpu.make_async_copy(
                table_hbm.at[idx_v.at[c0 + b]], rows_v.at[b], gsem
            ).wait()
            pltpu.async_copy(
                rows_v.at[b],
                out_hbm.at[pl.ds(base_w + (c0 + b) * CHUNK, CHUNK)],
                osem,
            )
        for b in range(NBUF):
            pltpu.make_async_copy(
                rows_v.at[b],
                out_hbm.at[pl.ds(base_w + (c0 + b) * CHUNK, CHUNK)],
                osem,
            ).wait()


_mesh = plsc.VectorSubcoreMesh(core_axis_name="c", subcore_axis_name="s")

_gather_call = pl.kernel(
    _body,
    out_type=jax.ShapeDtypeStruct((TOTAL, DIM), jnp.float32),
    mesh=_mesh,
    scratch_types=[
        pltpu.VMEM((NCHUNK, CHUNK), jnp.int32),
        pltpu.VMEM((NBUF, CHUNK, DIM), jnp.float32),
        pltpu.SemaphoreType.DMA,
        pltpu.SemaphoreType.DMA,
    ],
    compiler_params=pltpu.CompilerParams(use_tc_tiling_on_sc=False),
)


@jax.jit
def _run(idx3, weight):
    return _gather_call(idx3, weight)


def kernel(gene_indices, weight):
    idx3 = jnp.asarray(gene_indices, jnp.int32).reshape(NW, NCHUNK, CHUNK)
    out = _run(idx3, weight)
    return out.reshape(B, L, DIM)
